# Initial kernel scaffold; baseline (speedup 1.0000x reference)
import jax
import jax.numpy as jnp
from jax import lax
from jax.experimental import pallas as pl
from jax.experimental.pallas import tpu as pltpu
from jax.experimental.pallas import tpu_sc as plsc

DIM = 64
N_EDGES = 819200

NC = 2
NS = 16
LANES = 16
NW = NC * NS
E_PER_W = N_EDGES // NW
CHUNK = 128
N_CHUNKS = E_PER_W // CHUNK


def _body(xp_hbm, xt_hbm, ep_hbm, et_hbm, out_hbm,
          idx_p, idx_t, rows_p, rows_t, out_v, sem_p, sem_t):
    wid = lax.axis_index("s") * NC + lax.axis_index("c")
    base = wid * E_PER_W

    pltpu.sync_copy(ep_hbm.at[wid], idx_p)
    pltpu.sync_copy(et_hbm.at[wid], idx_t)

    def chunk_body(c):
        cp_p = pltpu.async_copy(xp_hbm.at[idx_p.at[c]], rows_p, sem_p)
        cp_t = pltpu.async_copy(xt_hbm.at[idx_t.at[c]], rows_t, sem_t)
        cp_p.wait()
        cp_t.wait()

        def edge_body(e):
            acc = rows_p[e, pl.ds(0, LANES)] * rows_t[e, pl.ds(0, LANES)]
            for q in range(1, DIM // LANES):
                acc += (rows_p[e, pl.ds(q * LANES, LANES)]
                        * rows_t[e, pl.ds(q * LANES, LANES)])
            out_v[e] = jnp.sum(acc)

        pl.loop(0, CHUNK)(edge_body)
        pltpu.sync_copy(out_v, out_hbm.at[pl.ds(base + c * CHUNK, CHUNK)])

    pl.loop(0, N_CHUNKS)(chunk_body)


@jax.jit
def kernel(x_playlist, x_track, edge_label_index):
    eidx = edge_label_index.astype(jnp.int32)
    ep = eidx[0].reshape(NW, N_CHUNKS, CHUNK)
    et = eidx[1].reshape(NW, N_CHUNKS, CHUNK)

    mesh = plsc.VectorSubcoreMesh(core_axis_name="c", subcore_axis_name="s")
    run = pl.kernel(
        _body,
        out_type=jax.ShapeDtypeStruct((N_EDGES,), jnp.float32),
        mesh=mesh,
        scratch_types=[
            pltpu.VMEM((N_CHUNKS, CHUNK), jnp.int32),
            pltpu.VMEM((N_CHUNKS, CHUNK), jnp.int32),
            pltpu.VMEM((CHUNK, DIM), jnp.float32),
            pltpu.VMEM((CHUNK, DIM), jnp.float32),
            pltpu.VMEM((CHUNK,), jnp.float32),
            pltpu.SemaphoreType.DMA,
            pltpu.SemaphoreType.DMA,
        ],
    )
    return run(x_playlist, x_track, ep, et)


# speedup vs baseline: 9.2927x; 9.2927x over previous
import jax
import jax.numpy as jnp
from jax import lax
from jax.experimental import pallas as pl
from jax.experimental.pallas import tpu as pltpu
from jax.experimental.pallas import tpu_sc as plsc

DIM = 64
N_EDGES = 819200

NC = 2
NS = 16
LANES = 16
NW = NC * NS
E_PER_W = N_EDGES // NW
CHUNK = 128
N_CHUNKS = E_PER_W // CHUNK


PAD = 17


def _body(xp_hbm, xt_hbm, ep_hbm, et_hbm, out_hbm,
          idx_p, idx_t, rows_p, rows_t, rsum, out_v, sem_p, sem_t):
    wid = lax.axis_index("s") * NC + lax.axis_index("c")
    base = wid * E_PER_W

    pltpu.sync_copy(ep_hbm.at[wid], idx_p)
    pltpu.sync_copy(et_hbm.at[wid], idx_t)

    iota = lax.iota(jnp.int32, LANES)
    iota_pad = iota * PAD

    def chunk_body(c):
        cp_p = pltpu.async_copy(xp_hbm.at[idx_p.at[c]], rows_p, sem_p)
        cp_t = pltpu.async_copy(xt_hbm.at[idx_t.at[c]], rows_t, sem_t)
        cp_p.wait()
        cp_t.wait()

        def edge_body(e):
            acc = rows_p[e, pl.ds(0, LANES)] * rows_t[e, pl.ds(0, LANES)]
            for q in range(1, DIM // LANES):
                acc += (rows_p[e, pl.ds(q * LANES, LANES)]
                        * rows_t[e, pl.ds(q * LANES, LANES)])
            plsc.store_scatter(rsum, [iota + e * PAD], acc)

        pl.loop(0, CHUNK)(edge_body)

        def group_body(g):
            gbase = g * (LANES * PAD)
            accg = plsc.load_gather(rsum, [iota_pad + gbase])
            for l in range(1, LANES):
                accg += plsc.load_gather(rsum, [iota_pad + (gbase + l)])
            out_v[pl.ds(g * LANES, LANES)] = accg

        pl.loop(0, CHUNK // LANES)(group_body)

        pltpu.sync_copy(out_v, out_hbm.at[pl.ds(base + c * CHUNK, CHUNK)])

    pl.loop(0, N_CHUNKS)(chunk_body)


@jax.jit
def kernel(x_playlist, x_track, edge_label_index):
    eidx = edge_label_index.astype(jnp.int32)
    ep = eidx[0].reshape(NW, N_CHUNKS, CHUNK)
    et = eidx[1].reshape(NW, N_CHUNKS, CHUNK)

    mesh = plsc.VectorSubcoreMesh(core_axis_name="c", subcore_axis_name="s")
    run = pl.kernel(
        _body,
        out_type=jax.ShapeDtypeStruct((N_EDGES,), jnp.float32),
        mesh=mesh,
        compiler_params=pltpu.CompilerParams(
            needs_layout_passes=False, use_tc_tiling_on_sc=False),
        scratch_types=[
            pltpu.VMEM((N_CHUNKS, CHUNK), jnp.int32),
            pltpu.VMEM((N_CHUNKS, CHUNK), jnp.int32),
            pltpu.VMEM((CHUNK, DIM), jnp.float32),
            pltpu.VMEM((CHUNK, DIM), jnp.float32),
            pltpu.VMEM((CHUNK * PAD,), jnp.float32),
            pltpu.VMEM((CHUNK,), jnp.float32),
            pltpu.SemaphoreType.DMA,
            pltpu.SemaphoreType.DMA,
        ],
    )
    return run(x_playlist, x_track, ep, et)


# speedup vs baseline: 18.5808x; 1.9995x over previous
import jax
import jax.numpy as jnp
from jax import lax
from jax.experimental import pallas as pl
from jax.experimental.pallas import tpu as pltpu
from jax.experimental.pallas import tpu_sc as plsc

DIM = 64
N_EDGES = 819200

NC = 2
NS = 16
LANES = 16
NW = NC * NS
E_PER_W = N_EDGES // NW
CHUNK = 128
N_CHUNKS = E_PER_W // CHUNK


PAD = 17
NBUF = 2


def _body(xp_hbm, xt_hbm, ep_hbm, et_hbm, out_hbm,
          idx_p, idx_t, rows_p, rows_t, rsum, out_all,
          sem_p0, sem_p1, sem_t0, sem_t1):
    wid = lax.axis_index("s") * NC + lax.axis_index("c")
    base = wid * E_PER_W
    sems_p = [sem_p0, sem_p1]
    sems_t = [sem_t0, sem_t1]

    pltpu.sync_copy(ep_hbm.at[wid], idx_p)
    pltpu.sync_copy(et_hbm.at[wid], idx_t)

    iota = lax.iota(jnp.int32, LANES)
    iota_pad = iota * PAD

    def fire(k, b):
        pltpu.async_copy(xp_hbm.at[idx_p.at[k]], rows_p.at[b], sems_p[b])
        pltpu.async_copy(xt_hbm.at[idx_t.at[k]], rows_t.at[b], sems_t[b])

    fire(0, 0)

    @pl.loop(0, N_CHUNKS, step=NBUF)
    def chunk_pair(c):
        for b in range(NBUF):
            k = c + b
            nb = (b + 1) % NBUF

            @pl.when(k + 1 < N_CHUNKS)
            def _():
                fire(k + 1, nb)

            pltpu.make_async_copy(
                xp_hbm.at[idx_p.at[k]], rows_p.at[b], sems_p[b]).wait()
            pltpu.make_async_copy(
                xt_hbm.at[idx_t.at[k]], rows_t.at[b], sems_t[b]).wait()

            rp = rows_p.at[b]
            rt = rows_t.at[b]

            @plsc.parallel_loop(0, CHUNK, unroll=8)
            def edge_body(e):
                acc = rp[e, pl.ds(0, LANES)] * rt[e, pl.ds(0, LANES)]
                for q in range(1, DIM // LANES):
                    acc += (rp[e, pl.ds(q * LANES, LANES)]
                            * rt[e, pl.ds(q * LANES, LANES)])
                plsc.store_scatter(rsum, [iota + e * PAD], acc)

            @plsc.parallel_loop(0, CHUNK // LANES, unroll=2)
            def group_body(g):
                gbase = g * (LANES * PAD)
                accg = plsc.load_gather(rsum, [iota_pad + gbase])
                for l in range(1, LANES):
                    accg += plsc.load_gather(rsum, [iota_pad + (gbase + l)])
                out_all[pl.ds(k * CHUNK + g * LANES, LANES)] = accg

    pltpu.sync_copy(out_all, out_hbm.at[pl.ds(base, E_PER_W)])


@jax.jit
def kernel(x_playlist, x_track, edge_label_index):
    eidx = edge_label_index.astype(jnp.int32)
    ep = eidx[0].reshape(NW, N_CHUNKS, CHUNK)
    et = eidx[1].reshape(NW, N_CHUNKS, CHUNK)

    mesh = plsc.VectorSubcoreMesh(core_axis_name="c", subcore_axis_name="s")
    run = pl.kernel(
        _body,
        out_type=jax.ShapeDtypeStruct((N_EDGES,), jnp.float32),
        mesh=mesh,
        compiler_params=pltpu.CompilerParams(
            needs_layout_passes=False, use_tc_tiling_on_sc=False),
        scratch_types=[
            pltpu.VMEM((N_CHUNKS, CHUNK), jnp.int32),
            pltpu.VMEM((N_CHUNKS, CHUNK), jnp.int32),
            pltpu.VMEM((NBUF, CHUNK, DIM), jnp.float32),
            pltpu.VMEM((NBUF, CHUNK, DIM), jnp.float32),
            pltpu.VMEM((CHUNK * PAD,), jnp.float32),
            pltpu.VMEM((E_PER_W,), jnp.float32),
            pltpu.SemaphoreType.DMA,
            pltpu.SemaphoreType.DMA,
            pltpu.SemaphoreType.DMA,
            pltpu.SemaphoreType.DMA,
        ],
    )
    return run(x_playlist, x_track, ep, et)


# speedup vs baseline: 18.6565x; 1.0041x over previous
import jax
import jax.numpy as jnp
from jax import lax
from jax.experimental import pallas as pl
from jax.experimental.pallas import tpu as pltpu
from jax.experimental.pallas import tpu_sc as plsc

DIM = 64
N_EDGES = 819200

NC = 2
NS = 16
LANES = 16
NW = NC * NS
E_PER_W = N_EDGES // NW
CHUNK = 128
N_CHUNKS = E_PER_W // CHUNK


PAD = 17
NBUF = 2


def _body(xp_hbm, xt_hbm, eidx_hbm, out_hbm,
          idx_p, idx_t, rows_p, rows_t, rsum, out_all,
          sem_p0, sem_p1, sem_t0, sem_t1):
    wid = lax.axis_index("s") * NC + lax.axis_index("c")
    base = wid * E_PER_W
    sems_p = [sem_p0, sem_p1]
    sems_t = [sem_t0, sem_t1]

    pltpu.sync_copy(eidx_hbm.at[0, pl.ds(base, E_PER_W)], idx_p)
    pltpu.sync_copy(eidx_hbm.at[1, pl.ds(base, E_PER_W)], idx_t)

    iota = lax.iota(jnp.int32, LANES)
    iota_pad = iota * PAD

    def fire(k, b):
        pltpu.async_copy(
            xp_hbm.at[idx_p.at[pl.ds(k * CHUNK, CHUNK)]],
            rows_p.at[b], sems_p[b])
        pltpu.async_copy(
            xt_hbm.at[idx_t.at[pl.ds(k * CHUNK, CHUNK)]],
            rows_t.at[b], sems_t[b])

    fire(0, 0)

    @pl.loop(0, N_CHUNKS, step=NBUF)
    def chunk_pair(c):
        for b in range(NBUF):
            k = c + b
            nb = (b + 1) % NBUF

            @pl.when(k + 1 < N_CHUNKS)
            def _():
                fire(k + 1, nb)

            pltpu.make_async_copy(
                xp_hbm.at[idx_p.at[pl.ds(k * CHUNK, CHUNK)]],
                rows_p.at[b], sems_p[b]).wait()
            pltpu.make_async_copy(
                xt_hbm.at[idx_t.at[pl.ds(k * CHUNK, CHUNK)]],
                rows_t.at[b], sems_t[b]).wait()

            rp = rows_p.at[b]
            rt = rows_t.at[b]

            @plsc.parallel_loop(0, CHUNK, unroll=8)
            def edge_body(e):
                acc = rp[e, pl.ds(0, LANES)] * rt[e, pl.ds(0, LANES)]
                for q in range(1, DIM // LANES):
                    acc += (rp[e, pl.ds(q * LANES, LANES)]
                            * rt[e, pl.ds(q * LANES, LANES)])
                plsc.store_scatter(rsum, [iota + e * PAD], acc)

            @plsc.parallel_loop(0, CHUNK // LANES, unroll=2)
            def group_body(g):
                gbase = g * (LANES * PAD)
                accg = plsc.load_gather(rsum, [iota_pad + gbase])
                for l in range(1, LANES):
                    accg += plsc.load_gather(rsum, [iota_pad + (gbase + l)])
                out_all[pl.ds(k * CHUNK + g * LANES, LANES)] = accg

    pltpu.sync_copy(out_all, out_hbm.at[pl.ds(base, E_PER_W)])


@jax.jit
def kernel(x_playlist, x_track, edge_label_index):
    eidx = edge_label_index.astype(jnp.int32)

    mesh = plsc.VectorSubcoreMesh(core_axis_name="c", subcore_axis_name="s")
    run = pl.kernel(
        _body,
        out_type=jax.ShapeDtypeStruct((N_EDGES,), jnp.float32),
        mesh=mesh,
        compiler_params=pltpu.CompilerParams(
            needs_layout_passes=False, use_tc_tiling_on_sc=False),
        scratch_types=[
            pltpu.VMEM((E_PER_W,), jnp.int32),
            pltpu.VMEM((E_PER_W,), jnp.int32),
            pltpu.VMEM((NBUF, CHUNK, DIM), jnp.float32),
            pltpu.VMEM((NBUF, CHUNK, DIM), jnp.float32),
            pltpu.VMEM((CHUNK * PAD,), jnp.float32),
            pltpu.VMEM((E_PER_W,), jnp.float32),
            pltpu.SemaphoreType.DMA,
            pltpu.SemaphoreType.DMA,
            pltpu.SemaphoreType.DMA,
            pltpu.SemaphoreType.DMA,
        ],
    )
    return run(x_playlist, x_track, eidx)


# speedup vs baseline: 19.6347x; 1.0524x over previous
import jax
import jax.numpy as jnp
from jax import lax
from jax.experimental import pallas as pl
from jax.experimental.pallas import tpu as pltpu
from jax.experimental.pallas import tpu_sc as plsc

DIM = 64
N_EDGES = 819200

NC = 2
NS = 16
LANES = 16
NW = NC * NS
E_PER_W = N_EDGES // NW
CHUNK = 128
N_CHUNKS = E_PER_W // CHUNK


PAD = 17
NBUF = 2


def _body(xp_hbm, xt_hbm, eidx_hbm, out_hbm,
          idx_p, idx_t, rows_p, rows_t, rsum, out_all,
          sem_p0, sem_p1, sem_t0, sem_t1):
    wid = lax.axis_index("s") * NC + lax.axis_index("c")
    base = wid * E_PER_W
    sems_p = [sem_p0, sem_p1]
    sems_t = [sem_t0, sem_t1]

    pltpu.sync_copy(eidx_hbm.at[0, pl.ds(base, E_PER_W)], idx_p)
    pltpu.sync_copy(eidx_hbm.at[1, pl.ds(base, E_PER_W)], idx_t)

    iota = lax.iota(jnp.int32, LANES)
    iota_pad = iota * PAD

    def fire(k, b):
        pltpu.async_copy(
            xp_hbm.at[idx_p.at[pl.ds(k * CHUNK, CHUNK)]],
            rows_p.at[b], sems_p[b])
        pltpu.async_copy(
            xt_hbm.at[idx_t.at[pl.ds(k * CHUNK, CHUNK)]],
            rows_t.at[b], sems_t[b])

    fire(0, 0)

    @pl.loop(0, N_CHUNKS, step=NBUF)
    def chunk_pair(c):
        for b in range(NBUF):
            k = c + b
            nb = (b + 1) % NBUF

            @pl.when(k + 1 < N_CHUNKS)
            def _():
                fire(k + 1, nb)

            pltpu.make_async_copy(
                xp_hbm.at[idx_p.at[pl.ds(k * CHUNK, CHUNK)]],
                rows_p.at[b], sems_p[b]).wait()
            pltpu.make_async_copy(
                xt_hbm.at[idx_t.at[pl.ds(k * CHUNK, CHUNK)]],
                rows_t.at[b], sems_t[b]).wait()

            rp = rows_p.at[b]
            rt = rows_t.at[b]

            @plsc.parallel_loop(0, CHUNK, unroll=8)
            def edge_body(e):
                prod = (rp[e, pl.ds(0, 2 * LANES)]
                        * rt[e, pl.ds(0, 2 * LANES)])
                prod += (rp[e, pl.ds(2 * LANES, 2 * LANES)]
                         * rt[e, pl.ds(2 * LANES, 2 * LANES)])
                lo, hi = plsc.unpack(prod, format=plsc.PackFormat.INTERLEAVED)
                acc = lo + hi
                plsc.store_scatter(rsum, [iota + e * PAD], acc)

            @plsc.parallel_loop(0, CHUNK // LANES, unroll=2)
            def group_body(g):
                gbase = g * (LANES * PAD)
                accg = plsc.load_gather(rsum, [iota_pad + gbase])
                for l in range(1, LANES):
                    accg += plsc.load_gather(rsum, [iota_pad + (gbase + l)])
                out_all[pl.ds(k * CHUNK + g * LANES, LANES)] = accg

    pltpu.sync_copy(out_all, out_hbm.at[pl.ds(base, E_PER_W)])


@jax.jit
def kernel(x_playlist, x_track, edge_label_index):
    eidx = edge_label_index.astype(jnp.int32)
    x_playlist = x_playlist.astype(jnp.bfloat16)
    x_track = x_track.astype(jnp.bfloat16)

    mesh = plsc.VectorSubcoreMesh(core_axis_name="c", subcore_axis_name="s")
    run = pl.kernel(
        _body,
        out_type=jax.ShapeDtypeStruct((N_EDGES,), jnp.float32),
        mesh=mesh,
        compiler_params=pltpu.CompilerParams(
            needs_layout_passes=False, use_tc_tiling_on_sc=False),
        scratch_types=[
            pltpu.VMEM((E_PER_W,), jnp.int32),
            pltpu.VMEM((E_PER_W,), jnp.int32),
            pltpu.VMEM((NBUF, CHUNK, DIM), jnp.bfloat16),
            pltpu.VMEM((NBUF, CHUNK, DIM), jnp.bfloat16),
            pltpu.VMEM((CHUNK * PAD,), jnp.float32),
            pltpu.VMEM((E_PER_W,), jnp.float32),
            pltpu.SemaphoreType.DMA,
            pltpu.SemaphoreType.DMA,
            pltpu.SemaphoreType.DMA,
            pltpu.SemaphoreType.DMA,
        ],
    )
    return run(x_playlist, x_track, eidx)
